# baseline (speedup 1.0000x reference)
import jax
import jax.numpy as jnp
from jax import lax
from jax.experimental import pallas as pl
from jax.experimental.pallas import tpu as pltpu
from jax.experimental.pallas import tpu_sc as plsc

NC = 2
NS = 16
L = 16
CH = 128

_f32 = jnp.float32


def _mesh():
    return plsc.VectorSubcoreMesh(
        core_axis_name="c", subcore_axis_name="s", num_cores=NC, num_subcores=NS
    )


def _zero_1d(ref, n):
    z = jnp.zeros((L,), _f32)

    def body(i, _):
        ref[pl.ds(i * L, L)] = z
        return 0

    lax.fori_loop(0, n // L, body, 0)


def _chunk_range(cid, sid, cpw0, cpw1):
    base = jnp.where(cid == 0, sid * cpw0, NS * cpw0 + sid * cpw1)
    cnt = jnp.where(cid == 0, cpw0, cpw1)
    return base, cnt


def _scalar_agg(Np, CPW0, CPW1, with_table):
    SL = Np // NS
    CPX = max(CPW0, CPW1)

    def body(*refs):
        if with_table:
            (src_hbm, dst_hbm, table_hbm, out_hbm,
             src_v, dst_v, table_v, acc_v, tmp_v, out_v, shared) = refs
        else:
            (dst_hbm, out_hbm, dst_v, acc_v, tmp_v, out_v, shared) = refs
        cid = lax.axis_index("c")
        sid = lax.axis_index("s")
        base, _ = _chunk_range(cid, sid, CPW0, CPW1)

        _zero_1d(acc_v, Np)
        pltpu.sync_copy(dst_hbm.at[pl.ds(base, CPX)], dst_v)
        if with_table:
            pltpu.sync_copy(src_hbm.at[pl.ds(base, CPX)], src_v)
            pltpu.sync_copy(table_hbm, table_v)
        ones = jnp.ones((L,), _f32)

        def edge_body(g, _):
            for r in range(CH // L):
                dv = dst_v[g, pl.ds(r * L, L)]
                if with_table:
                    sv = src_v[g, pl.ds(r * L, L)]
                    vals = plsc.load_gather(table_v, [sv])
                else:
                    vals = ones
                plsc.addupdate_scatter(acc_v, [dv], vals)
            return 0

        @pl.when(cid == 0)
        def _():
            lax.fori_loop(0, CPW0, edge_body, 0)

        @pl.when(cid != 0)
        def _():
            lax.fori_loop(0, CPW1, edge_body, 0)

        pltpu.sync_copy(acc_v, shared.at[sid])
        plsc.subcore_barrier()
        for t in range(NS):
            pltpu.sync_copy(shared.at[t, pl.ds(sid * SL, SL)], tmp_v.at[t])

        def merge_body(j, _):
            s = tmp_v[0, pl.ds(j * L, L)]
            for t in range(1, NS):
                s = s + tmp_v[t, pl.ds(j * L, L)]
            out_v[pl.ds(j * L, L)] = s
            return 0

        lax.fori_loop(0, SL // L, merge_body, 0)
        pltpu.sync_copy(out_v, out_hbm.at[cid, pl.ds(sid * SL, SL)])

    scratch = []
    if with_table:
        scratch.append(pltpu.VMEM((CPX, CH), jnp.int32))
    scratch.append(pltpu.VMEM((CPX, CH), jnp.int32))
    if with_table:
        scratch.append(pltpu.VMEM((Np,), _f32))
    scratch += [
        pltpu.VMEM((Np,), _f32),
        pltpu.VMEM((NS, SL), _f32),
        pltpu.VMEM((SL,), _f32),
        pltpu.VMEM_SHARED((NS, Np), _f32),
    ]
    return pl.kernel(
        body,
        out_type=jax.ShapeDtypeStruct((NC, Np), _f32),
        mesh=_mesh(),
        scratch_types=scratch,
        compiler_params=pltpu.CompilerParams(
            needs_layout_passes=False, use_tc_tiling_on_sc=False
        ),
    )


NB = 3


def _row_agg(Np, D, CPW0, CPW1):
    SL = Np // NS
    RB = SL // CH
    RING = 2 * NB
    assert CPW0 % RING == 0 and CPW1 % RING == 0 and min(CPW0, CPW1) >= RING
    CPX = max(CPW0, CPW1)

    def body(y_hbm, src_hbm, dst_hbm, out_hbm,
             src_v, dst_v, rows, gsems, ssems, shared):
        cid = lax.axis_index("c")
        sid = lax.axis_index("s")
        base, _ = _chunk_range(cid, sid, CPW0, CPW1)

        pltpu.sync_copy(src_hbm.at[pl.ds(base, CPX)], src_v)
        pltpu.sync_copy(dst_hbm.at[pl.ds(base, CPX)], dst_v)

        zrow = jnp.zeros((L,), _f32)

        def zbody(i, _):
            for j in range(D // L):
                rows.at[0][i, pl.ds(j * L, L)] = zrow
            return 0

        lax.fori_loop(0, CH, zbody, 0)
        for k in range(RB):
            pltpu.sync_copy(rows.at[0], shared.at[pl.ds(sid * SL + k * CH, CH)])
        plsc.subcore_barrier()

        def run_ring(cpw):
            for b in range(NB):
                pltpu.async_copy(y_hbm.at[src_v.at[b]], rows.at[b], gsems.at[b])

            def edge_body(g, _):
                for b in range(RING):
                    c = g * RING + b
                    b2 = (b + NB) % RING
                    pltpu.make_async_copy(
                        y_hbm.at[src_v.at[c]], rows.at[b], gsems.at[b]
                    ).wait()
                    pltpu.async_copy(
                        rows.at[b], shared.at[dst_v.at[c]], ssems.at[b], add=True
                    )

                    @pl.when(c >= NB)
                    def _():
                        pltpu.make_async_copy(
                            rows.at[b2], shared.at[dst_v.at[c - NB]], ssems.at[b2]
                        ).wait()

                    @pl.when(c + NB < cpw)
                    def _():
                        pltpu.async_copy(
                            y_hbm.at[src_v.at[c + NB]], rows.at[b2], gsems.at[b2]
                        )
                return 0

            lax.fori_loop(0, cpw // RING, edge_body, 0)
            for k in range(NB):
                c = cpw - NB + k
                b2 = c % RING
                pltpu.make_async_copy(
                    rows.at[b2], shared.at[dst_v.at[c]], ssems.at[b2]
                ).wait()

        @pl.when(cid == 0)
        def _():
            run_ring(CPW0)

        @pl.when(cid != 0)
        def _():
            run_ring(CPW1)

        plsc.subcore_barrier()

        for k in range(RB):
            pltpu.sync_copy(shared.at[pl.ds(sid * SL + k * CH, CH)], rows.at[0])
            pltpu.sync_copy(rows.at[0], out_hbm.at[cid, pl.ds(sid * SL + k * CH, CH)])

    return pl.kernel(
        body,
        out_type=jax.ShapeDtypeStruct((NC, Np, D), _f32),
        mesh=_mesh(),
        scratch_types=[
            pltpu.VMEM((CPX, CH), jnp.int32),
            pltpu.VMEM((CPX, CH), jnp.int32),
            pltpu.VMEM((2 * NB, CH, D), _f32),
            pltpu.SemaphoreType.DMA((2 * NB,)),
            pltpu.SemaphoreType.DMA((2 * NB,)),
            pltpu.VMEM_SHARED((Np, D), _f32),
        ],
        compiler_params=pltpu.CompilerParams(
            needs_layout_passes=False, use_tc_tiling_on_sc=False
        ),
    )


def _dis(deg_ref):
    deg = deg_ref[:, 0:1] + deg_ref[:, 1:2] + 1.0
    return lax.rsqrt(deg)


def _tc_y1(x_p, W1, degp_t, RB=2048):
    Np, DI = x_p.shape
    DH = W1.shape[1]

    def body(x_ref, w_ref, deg_ref, y_ref):
        dis = _dis(deg_ref)
        xw = jnp.dot(x_ref[...], w_ref[...], preferred_element_type=_f32)
        y_ref[...] = dis * xw

    return pl.pallas_call(
        body,
        grid=(Np // RB,),
        in_specs=[
            pl.BlockSpec((RB, DI), lambda i: (i, 0)),
            pl.BlockSpec((DI, DH), lambda i: (0, 0)),
            pl.BlockSpec((RB, 2), lambda i: (i, 0)),
        ],
        out_specs=pl.BlockSpec((RB, DH), lambda i: (i, 0)),
        out_shape=jax.ShapeDtypeStruct((Np, DH), _f32),
    )(x_p, W1, degp_t)


def _tc_y2(y1, aggp, degp_t, W2, b1, RB=2048):
    Np, DH = y1.shape

    def body(y_ref, agg_ref, deg_ref, w2_ref, b1_ref, y2_ref):
        dis = _dis(deg_ref)
        agg = agg_ref[0] + agg_ref[1]
        h = jnp.maximum(dis * (agg + y_ref[...]) + b1_ref[...], 0.0)
        z = jnp.dot(h, w2_ref[...], preferred_element_type=_f32)
        y2_ref[...] = dis * z

    return pl.pallas_call(
        body,
        grid=(Np // RB,),
        in_specs=[
            pl.BlockSpec((RB, DH), lambda i: (i, 0)),
            pl.BlockSpec((NC, RB, DH), lambda i: (0, i, 0)),
            pl.BlockSpec((RB, 2), lambda i: (i, 0)),
            pl.BlockSpec((DH, 1), lambda i: (0, 0)),
            pl.BlockSpec((1, DH), lambda i: (0, 0)),
        ],
        out_specs=pl.BlockSpec((RB, 1), lambda i: (i, 0)),
        out_shape=jax.ShapeDtypeStruct((Np, 1), _f32),
    )(y1, aggp, degp_t, W2, b1)


def _tc_out(agg2p_t, y2, degp_t, b2, RB=2048):
    Np = y2.shape[0]

    def body(a2_ref, y2_ref, deg_ref, b2_ref, o_ref):
        dis = _dis(deg_ref)
        agg2 = a2_ref[:, 0:1] + a2_ref[:, 1:2]
        o_ref[...] = dis * (agg2 + y2_ref[...]) + b2_ref[...]

    return pl.pallas_call(
        body,
        grid=(Np // RB,),
        in_specs=[
            pl.BlockSpec((RB, 2), lambda i: (i, 0)),
            pl.BlockSpec((RB, 1), lambda i: (i, 0)),
            pl.BlockSpec((RB, 2), lambda i: (i, 0)),
            pl.BlockSpec((1, 1), lambda i: (0, 0)),
        ],
        out_specs=pl.BlockSpec((RB, 1), lambda i: (i, 0)),
        out_shape=jax.ShapeDtypeStruct((Np, 1), _f32),
    )(agg2p_t, y2, degp_t, b2)


@jax.jit
def kernel(x, edge_index, W1, b1, W2, b2):
    N, DI = x.shape
    DH = W1.shape[1]
    E = edge_index.shape[1]

    Np = ((N + 1 + NS * L - 1) // (NS * L)) * (NS * L)
    RING = 2 * NB
    nch = (E + CH - 1) // CH
    PW = (nch + NS - 1) // NS
    PW = (PW + RING - 1) // RING * RING
    TCH = NS * PW
    Ep = TCH * CH
    RCPW0 = max(RING, PW // 2 // RING * RING)
    RCPW1 = PW - RCPW0
    SCPW0 = max(1, PW // 2)
    SCPW1 = PW - SCPW0

    pad = jnp.full((Ep - E,), N, jnp.int32)
    src2 = jnp.concatenate([edge_index[0], pad]).reshape(TCH, CH)
    dst2 = jnp.concatenate([edge_index[1], pad]).reshape(TCH, CH)
    x_p = jnp.pad(x, ((0, Np - N), (0, 0)))

    degp = _scalar_agg(Np, SCPW1, SCPW0, with_table=False)(dst2)
    degp_t = degp.T
    y1 = _tc_y1(x_p, W1, degp_t)
    aggp = _row_agg(Np, DH, RCPW1, RCPW0)(y1, src2, dst2)
    y2 = _tc_y2(y1, aggp, degp_t, W2, b1.reshape(1, DH))
    agg2p = _scalar_agg(Np, SCPW1, SCPW0, with_table=True)(
        src2, dst2, y2.reshape(Np))
    out = _tc_out(agg2p.T, y2, degp_t, b2.reshape(1, 1))
    return out[:N]


# speedup vs baseline: 1.8032x; 1.8032x over previous
import jax
import jax.numpy as jnp
from jax import lax
from jax.experimental import pallas as pl
from jax.experimental.pallas import tpu as pltpu
from jax.experimental.pallas import tpu_sc as plsc

NC = 2
NS = 16
L = 16
CH = 128

_f32 = jnp.float32


def _mesh():
    return plsc.VectorSubcoreMesh(
        core_axis_name="c", subcore_axis_name="s", num_cores=NC, num_subcores=NS
    )


def _zero_1d(ref, n):
    z = jnp.zeros((L,), _f32)

    def body(i, _):
        ref[pl.ds(i * L, L)] = z
        return 0

    lax.fori_loop(0, n // L, body, 0)


def _chunk_range(cid, sid, cpw0, cpw1):
    base = jnp.where(cid == 0, sid * cpw0, NS * cpw0 + sid * cpw1)
    cnt = jnp.where(cid == 0, cpw0, cpw1)
    return base, cnt


def _scalar_agg(Np, CPW0, CPW1, with_table):
    SL = Np // NS
    CPX = max(CPW0, CPW1)

    def body(*refs):
        if with_table:
            (src_hbm, dst_hbm, table_hbm, out_hbm,
             src_v, dst_v, table_v, acc_v, tmp_v, out_v, shared) = refs
        else:
            (dst_hbm, out_hbm, dst_v, acc_v, tmp_v, out_v, shared) = refs
        cid = lax.axis_index("c")
        sid = lax.axis_index("s")
        base, _ = _chunk_range(cid, sid, CPW0, CPW1)

        _zero_1d(acc_v, Np)
        pltpu.sync_copy(dst_hbm.at[pl.ds(base, CPX)], dst_v)
        if with_table:
            pltpu.sync_copy(src_hbm.at[pl.ds(base, CPX)], src_v)
            pltpu.sync_copy(table_hbm, table_v)
        ones = jnp.ones((L,), _f32)

        def edge_body(g, _):
            for r in range(CH // L):
                dv = dst_v[g, pl.ds(r * L, L)]
                if with_table:
                    sv = src_v[g, pl.ds(r * L, L)]
                    vals = plsc.load_gather(table_v, [sv])
                else:
                    vals = ones
                plsc.addupdate_scatter(acc_v, [dv], vals)
            return 0

        @pl.when(cid == 0)
        def _():
            lax.fori_loop(0, CPW0, edge_body, 0)

        @pl.when(cid != 0)
        def _():
            lax.fori_loop(0, CPW1, edge_body, 0)

        pltpu.sync_copy(acc_v, shared.at[sid])
        plsc.subcore_barrier()
        for t in range(NS):
            pltpu.sync_copy(shared.at[t, pl.ds(sid * SL, SL)], tmp_v.at[t])

        def merge_body(j, _):
            s = tmp_v[0, pl.ds(j * L, L)]
            for t in range(1, NS):
                s = s + tmp_v[t, pl.ds(j * L, L)]
            out_v[pl.ds(j * L, L)] = s
            return 0

        lax.fori_loop(0, SL // L, merge_body, 0)
        pltpu.sync_copy(out_v, out_hbm.at[cid, pl.ds(sid * SL, SL)])

    scratch = []
    if with_table:
        scratch.append(pltpu.VMEM((CPX, CH), jnp.int32))
    scratch.append(pltpu.VMEM((CPX, CH), jnp.int32))
    if with_table:
        scratch.append(pltpu.VMEM((Np,), _f32))
    scratch += [
        pltpu.VMEM((Np,), _f32),
        pltpu.VMEM((NS, SL), _f32),
        pltpu.VMEM((SL,), _f32),
        pltpu.VMEM_SHARED((NS, Np), _f32),
    ]
    return pl.kernel(
        body,
        out_type=jax.ShapeDtypeStruct((NC, Np), _f32),
        mesh=_mesh(),
        scratch_types=scratch,
        compiler_params=pltpu.CompilerParams(
            needs_layout_passes=False, use_tc_tiling_on_sc=False
        ),
    )


NB = 1


def _row_agg(Np, D, CPW0, CPW1):
    SL = Np // NS
    RB = SL // CH
    RING = 2 * NB
    assert CPW0 % RING == 0 and CPW1 % RING == 0 and min(CPW0, CPW1) >= RING
    CPX = max(CPW0, CPW1)

    def body(y_hbm, src_hbm, dst_hbm, out_hbm,
             src_v, dst_v, rows, gsems, ssems, shared):
        cid = lax.axis_index("c")
        sid = lax.axis_index("s")
        base, _ = _chunk_range(cid, sid, CPW0, CPW1)

        pltpu.sync_copy(src_hbm.at[pl.ds(base, CPX)], src_v)
        pltpu.sync_copy(dst_hbm.at[pl.ds(base, CPX)], dst_v)

        zrow = jnp.zeros((L,), _f32)

        def zbody(i, _):
            for j in range(D // L):
                rows.at[0][i, pl.ds(j * L, L)] = zrow
            return 0

        lax.fori_loop(0, CH, zbody, 0)
        for k in range(RB):
            pltpu.sync_copy(rows.at[0], shared.at[pl.ds(sid * SL + k * CH, CH)])
        plsc.subcore_barrier()

        def run_ring(cpw):
            for b in range(NB):
                pltpu.async_copy(y_hbm.at[src_v.at[b]], rows.at[b], gsems.at[b])

            def edge_body(g, _):
                for b in range(RING):
                    c = g * RING + b
                    b2 = (b + NB) % RING
                    pltpu.make_async_copy(
                        y_hbm.at[src_v.at[c]], rows.at[b], gsems.at[b]
                    ).wait()
                    pltpu.async_copy(
                        rows.at[b], shared.at[dst_v.at[c]], ssems.at[b], add=True
                    )

                    @pl.when(c >= NB)
                    def _():
                        pltpu.make_async_copy(
                            rows.at[b2], shared.at[dst_v.at[c - NB]], ssems.at[b2]
                        ).wait()

                    @pl.when(c + NB < cpw)
                    def _():
                        pltpu.async_copy(
                            y_hbm.at[src_v.at[c + NB]], rows.at[b2], gsems.at[b2]
                        )
                return 0

            lax.fori_loop(0, cpw // RING, edge_body, 0)
            for k in range(NB):
                c = cpw - NB + k
                b2 = c % RING
                pltpu.make_async_copy(
                    rows.at[b2], shared.at[dst_v.at[c]], ssems.at[b2]
                ).wait()

        @pl.when(cid == 0)
        def _():
            run_ring(CPW0)

        @pl.when(cid != 0)
        def _():
            run_ring(CPW1)

        plsc.subcore_barrier()

        for k in range(RB):
            pltpu.sync_copy(shared.at[pl.ds(sid * SL + k * CH, CH)], rows.at[0])
            pltpu.sync_copy(rows.at[0], out_hbm.at[cid, pl.ds(sid * SL + k * CH, CH)])

    return pl.kernel(
        body,
        out_type=jax.ShapeDtypeStruct((NC, Np, D), _f32),
        mesh=_mesh(),
        scratch_types=[
            pltpu.VMEM((CPX, CH), jnp.int32),
            pltpu.VMEM((CPX, CH), jnp.int32),
            pltpu.VMEM((2 * NB, CH, D), _f32),
            pltpu.SemaphoreType.DMA((2 * NB,)),
            pltpu.SemaphoreType.DMA((2 * NB,)),
            pltpu.VMEM_SHARED((Np, D), _f32),
        ],
        compiler_params=pltpu.CompilerParams(
            needs_layout_passes=False, use_tc_tiling_on_sc=False
        ),
    )


def _dis(deg_ref):
    deg = deg_ref[:, 0:1] + deg_ref[:, 1:2] + 1.0
    return lax.rsqrt(deg)


def _tc_y1(x_p, W1, degp_t, RB=2048):
    Np, DI = x_p.shape
    DH = W1.shape[1]

    def body(x_ref, w_ref, deg_ref, y_ref):
        dis = _dis(deg_ref)
        xw = jnp.dot(x_ref[...], w_ref[...], preferred_element_type=_f32)
        y_ref[...] = dis * xw

    return pl.pallas_call(
        body,
        grid=(Np // RB,),
        in_specs=[
            pl.BlockSpec((RB, DI), lambda i: (i, 0)),
            pl.BlockSpec((DI, DH), lambda i: (0, 0)),
            pl.BlockSpec((RB, 2), lambda i: (i, 0)),
        ],
        out_specs=pl.BlockSpec((RB, DH), lambda i: (i, 0)),
        out_shape=jax.ShapeDtypeStruct((Np, DH), _f32),
    )(x_p, W1, degp_t)


def _tc_y2(y1, aggp, degp_t, W2, b1, RB=2048):
    Np, DH = y1.shape

    def body(y_ref, agg_ref, deg_ref, w2_ref, b1_ref, y2_ref):
        dis = _dis(deg_ref)
        agg = agg_ref[0] + agg_ref[1]
        h = jnp.maximum(dis * (agg + y_ref[...]) + b1_ref[...], 0.0)
        z = jnp.dot(h, w2_ref[...], preferred_element_type=_f32)
        y2_ref[...] = dis * z

    return pl.pallas_call(
        body,
        grid=(Np // RB,),
        in_specs=[
            pl.BlockSpec((RB, DH), lambda i: (i, 0)),
            pl.BlockSpec((NC, RB, DH), lambda i: (0, i, 0)),
            pl.BlockSpec((RB, 2), lambda i: (i, 0)),
            pl.BlockSpec((DH, 1), lambda i: (0, 0)),
            pl.BlockSpec((1, DH), lambda i: (0, 0)),
        ],
        out_specs=pl.BlockSpec((RB, 1), lambda i: (i, 0)),
        out_shape=jax.ShapeDtypeStruct((Np, 1), _f32),
    )(y1, aggp, degp_t, W2, b1)


def _tc_out(agg2p_t, y2, degp_t, b2, RB=2048):
    Np = y2.shape[0]

    def body(a2_ref, y2_ref, deg_ref, b2_ref, o_ref):
        dis = _dis(deg_ref)
        agg2 = a2_ref[:, 0:1] + a2_ref[:, 1:2]
        o_ref[...] = dis * (agg2 + y2_ref[...]) + b2_ref[...]

    return pl.pallas_call(
        body,
        grid=(Np // RB,),
        in_specs=[
            pl.BlockSpec((RB, 2), lambda i: (i, 0)),
            pl.BlockSpec((RB, 1), lambda i: (i, 0)),
            pl.BlockSpec((RB, 2), lambda i: (i, 0)),
            pl.BlockSpec((1, 1), lambda i: (0, 0)),
        ],
        out_specs=pl.BlockSpec((RB, 1), lambda i: (i, 0)),
        out_shape=jax.ShapeDtypeStruct((Np, 1), _f32),
    )(agg2p_t, y2, degp_t, b2)


@jax.jit
def kernel(x, edge_index, W1, b1, W2, b2):
    N, DI = x.shape
    DH = W1.shape[1]
    E = edge_index.shape[1]

    Np = ((N + 1 + NS * L - 1) // (NS * L)) * (NS * L)
    RING = 2 * NB
    nch = (E + CH - 1) // CH
    PW = (nch + NS - 1) // NS
    PW = (PW + RING - 1) // RING * RING
    TCH = NS * PW
    Ep = TCH * CH
    RCPW0 = max(RING, (PW * 22 // 100 + RING - 1) // RING * RING)
    RCPW1 = PW - RCPW0
    SCPW0 = max(1, PW * 4 // 10)
    SCPW1 = PW - SCPW0

    pad = jnp.full((Ep - E,), N, jnp.int32)
    src2 = jnp.concatenate([edge_index[0], pad]).reshape(TCH, CH)
    dst2 = jnp.concatenate([edge_index[1], pad]).reshape(TCH, CH)
    x_p = jnp.pad(x, ((0, Np - N), (0, 0)))

    degp = _scalar_agg(Np, SCPW1, SCPW0, with_table=False)(dst2)
    degp_t = degp.T
    y1 = _tc_y1(x_p, W1, degp_t)
    aggp = _row_agg(Np, DH, RCPW1, RCPW0)(y1, src2, dst2)
    y2 = _tc_y2(y1, aggp, degp_t, W2, b1.reshape(1, DH))
    agg2p = _scalar_agg(Np, SCPW1, SCPW0, with_table=True)(
        src2, dst2, y2.reshape(Np))
    out = _tc_out(agg2p.T, y2, degp_t, b2.reshape(1, 1))
    return out[:N]


# speedup vs baseline: 1.9061x; 1.0570x over previous
import jax
import jax.numpy as jnp
from jax import lax
from jax.experimental import pallas as pl
from jax.experimental.pallas import tpu as pltpu
from jax.experimental.pallas import tpu_sc as plsc

NC = 2
NS = 16
L = 16
CH = 128

_f32 = jnp.float32


def _mesh():
    return plsc.VectorSubcoreMesh(
        core_axis_name="c", subcore_axis_name="s", num_cores=NC, num_subcores=NS
    )


def _zero_1d(ref, n):
    z = jnp.zeros((L,), _f32)

    def body(i, _):
        ref[pl.ds(i * L, L)] = z
        return 0

    lax.fori_loop(0, n // L, body, 0)


def _chunk_range(cid, sid, cpw0, cpw1):
    base = jnp.where(cid == 0, sid * cpw0, NS * cpw0 + sid * cpw1)
    cnt = jnp.where(cid == 0, cpw0, cpw1)
    return base, cnt


def _scalar_agg(Np, CPW0, CPW1, with_table):
    SL = Np // NS
    CPX = max(CPW0, CPW1)

    def body(*refs):
        if with_table:
            (src_hbm, dst_hbm, table_hbm, out_hbm,
             src_v, dst_v, table_v, acc_v, tmp_v, out_v, shared) = refs
        else:
            (dst_hbm, out_hbm, dst_v, acc_v, tmp_v, out_v, shared) = refs
        cid = lax.axis_index("c")
        sid = lax.axis_index("s")
        base, _ = _chunk_range(cid, sid, CPW0, CPW1)

        _zero_1d(acc_v, Np)
        pltpu.sync_copy(dst_hbm.at[pl.ds(base, CPX)], dst_v)
        if with_table:
            pltpu.sync_copy(src_hbm.at[pl.ds(base, CPX)], src_v)
            pltpu.sync_copy(table_hbm, table_v)
        ones = jnp.ones((L,), _f32)

        def edge_body(g, _):
            for r in range(CH // L):
                dv = dst_v[g, pl.ds(r * L, L)]
                if with_table:
                    sv = src_v[g, pl.ds(r * L, L)]
                    vals = plsc.load_gather(table_v, [sv])
                else:
                    vals = ones
                plsc.addupdate_scatter(acc_v, [dv], vals)
            return 0

        @pl.when(cid == 0)
        def _():
            lax.fori_loop(0, CPW0, edge_body, 0)

        @pl.when(cid != 0)
        def _():
            lax.fori_loop(0, CPW1, edge_body, 0)

        pltpu.sync_copy(acc_v, shared.at[sid])
        plsc.subcore_barrier()
        for t in range(NS):
            pltpu.sync_copy(shared.at[t, pl.ds(sid * SL, SL)], tmp_v.at[t])

        def merge_body(j, _):
            s = tmp_v[0, pl.ds(j * L, L)]
            for t in range(1, NS):
                s = s + tmp_v[t, pl.ds(j * L, L)]
            out_v[pl.ds(j * L, L)] = s
            return 0

        lax.fori_loop(0, SL // L, merge_body, 0)
        pltpu.sync_copy(out_v, out_hbm.at[cid, pl.ds(sid * SL, SL)])

    scratch = []
    if with_table:
        scratch.append(pltpu.VMEM((CPX, CH), jnp.int32))
    scratch.append(pltpu.VMEM((CPX, CH), jnp.int32))
    if with_table:
        scratch.append(pltpu.VMEM((Np,), _f32))
    scratch += [
        pltpu.VMEM((Np,), _f32),
        pltpu.VMEM((NS, SL), _f32),
        pltpu.VMEM((SL,), _f32),
        pltpu.VMEM_SHARED((NS, Np), _f32),
    ]
    return pl.kernel(
        body,
        out_type=jax.ShapeDtypeStruct((NC, Np), _f32),
        mesh=_mesh(),
        scratch_types=scratch,
        compiler_params=pltpu.CompilerParams(
            needs_layout_passes=False, use_tc_tiling_on_sc=False
        ),
    )


NB = 1


def _row_agg(Np, D, CPW0, CPW1):
    SL = Np // NS
    RB = SL // CH
    RING = 2 * NB
    assert CPW0 % RING == 0 and CPW1 % RING == 0 and min(CPW0, CPW1) >= RING
    CPX = max(CPW0, CPW1)

    def body(y_hbm, src_hbm, dst_hbm, out_hbm,
             src_v, dst_v, rows, gsems, ssems, shared):
        cid = lax.axis_index("c")
        sid = lax.axis_index("s")
        base, _ = _chunk_range(cid, sid, CPW0, CPW1)

        pltpu.sync_copy(src_hbm.at[pl.ds(base, CPX)], src_v)
        pltpu.sync_copy(dst_hbm.at[pl.ds(base, CPX)], dst_v)

        zrow = jnp.zeros((L,), _f32)

        def zbody(i, _):
            for j in range(D // L):
                rows.at[0][i, pl.ds(j * L, L)] = zrow
            return 0

        lax.fori_loop(0, CH, zbody, 0)
        for k in range(RB):
            pltpu.sync_copy(rows.at[0], shared.at[pl.ds(sid * SL + k * CH, CH)])
        plsc.subcore_barrier()

        def run_ring(cpw):
            for b in range(NB):
                pltpu.async_copy(y_hbm.at[src_v.at[b]], rows.at[b], gsems.at[b])

            def edge_body(g, _):
                for b in range(RING):
                    c = g * RING + b
                    b2 = (b + NB) % RING
                    pltpu.make_async_copy(
                        y_hbm.at[src_v.at[c]], rows.at[b], gsems.at[b]
                    ).wait()
                    pltpu.async_copy(
                        rows.at[b], shared.at[dst_v.at[c]], ssems.at[b], add=True
                    )

                    @pl.when(c >= NB)
                    def _():
                        pltpu.make_async_copy(
                            rows.at[b2], shared.at[dst_v.at[c - NB]], ssems.at[b2]
                        ).wait()

                    @pl.when(c + NB < cpw)
                    def _():
                        pltpu.async_copy(
                            y_hbm.at[src_v.at[c + NB]], rows.at[b2], gsems.at[b2]
                        )
                return 0

            lax.fori_loop(0, cpw // RING, edge_body, 0)
            for k in range(NB):
                c = cpw - NB + k
                b2 = c % RING
                pltpu.make_async_copy(
                    rows.at[b2], shared.at[dst_v.at[c]], ssems.at[b2]
                ).wait()

        @pl.when(cid == 0)
        def _():
            run_ring(CPW0)

        @pl.when(cid != 0)
        def _():
            run_ring(CPW1)

        plsc.subcore_barrier()

        for k in range(RB):
            pltpu.sync_copy(shared.at[pl.ds(sid * SL + k * CH, CH)], rows.at[0])
            pltpu.sync_copy(rows.at[0], out_hbm.at[cid, pl.ds(sid * SL + k * CH, CH)])

    return pl.kernel(
        body,
        out_type=jax.ShapeDtypeStruct((NC, Np, D), _f32),
        mesh=_mesh(),
        scratch_types=[
            pltpu.VMEM((CPX, CH), jnp.int32),
            pltpu.VMEM((CPX, CH), jnp.int32),
            pltpu.VMEM((2 * NB, CH, D), _f32),
            pltpu.SemaphoreType.DMA((2 * NB,)),
            pltpu.SemaphoreType.DMA((2 * NB,)),
            pltpu.VMEM_SHARED((Np, D), _f32),
        ],
        compiler_params=pltpu.CompilerParams(
            needs_layout_passes=False, use_tc_tiling_on_sc=False
        ),
    )


def _dis(deg_ref):
    deg = deg_ref[:, 0:1] + deg_ref[:, 1:2] + 1.0
    return lax.rsqrt(deg)


def _tc_y1(x_p, W1, degp_t, RB=2048):
    Np, DI = x_p.shape
    DH = W1.shape[1]

    def body(x_ref, w_ref, deg_ref, y_ref):
        dis = _dis(deg_ref)
        xw = jnp.dot(x_ref[...], w_ref[...], preferred_element_type=_f32)
        y_ref[...] = dis * xw

    return pl.pallas_call(
        body,
        grid=(Np // RB,),
        in_specs=[
            pl.BlockSpec((RB, DI), lambda i: (i, 0)),
            pl.BlockSpec((DI, DH), lambda i: (0, 0)),
            pl.BlockSpec((RB, 2), lambda i: (i, 0)),
        ],
        out_specs=pl.BlockSpec((RB, DH), lambda i: (i, 0)),
        out_shape=jax.ShapeDtypeStruct((Np, DH), _f32),
    )(x_p, W1, degp_t)


def _tc_y2(y1, aggp, degp_t, W2, b1, RB=2048):
    Np, DH = y1.shape

    def body(y_ref, agg_ref, deg_ref, w2_ref, b1_ref, y2_ref):
        dis = _dis(deg_ref)
        agg = agg_ref[0] + agg_ref[1]
        h = jnp.maximum(dis * (agg + y_ref[...]) + b1_ref[...], 0.0)
        z = jnp.dot(h, w2_ref[...], preferred_element_type=_f32)
        y2_ref[...] = dis * z

    return pl.pallas_call(
        body,
        grid=(Np // RB,),
        in_specs=[
            pl.BlockSpec((RB, DH), lambda i: (i, 0)),
            pl.BlockSpec((NC, RB, DH), lambda i: (0, i, 0)),
            pl.BlockSpec((RB, 2), lambda i: (i, 0)),
            pl.BlockSpec((DH, 1), lambda i: (0, 0)),
            pl.BlockSpec((1, DH), lambda i: (0, 0)),
        ],
        out_specs=pl.BlockSpec((RB, 1), lambda i: (i, 0)),
        out_shape=jax.ShapeDtypeStruct((Np, 1), _f32),
    )(y1, aggp, degp_t, W2, b1)


def _tc_out(agg2p_t, y2, degp_t, b2, RB=2048):
    Np = y2.shape[0]

    def body(a2_ref, y2_ref, deg_ref, b2_ref, o_ref):
        dis = _dis(deg_ref)
        agg2 = a2_ref[:, 0:1] + a2_ref[:, 1:2]
        o_ref[...] = dis * (agg2 + y2_ref[...]) + b2_ref[...]

    return pl.pallas_call(
        body,
        grid=(Np // RB,),
        in_specs=[
            pl.BlockSpec((RB, 2), lambda i: (i, 0)),
            pl.BlockSpec((RB, 1), lambda i: (i, 0)),
            pl.BlockSpec((RB, 2), lambda i: (i, 0)),
            pl.BlockSpec((1, 1), lambda i: (0, 0)),
        ],
        out_specs=pl.BlockSpec((RB, 1), lambda i: (i, 0)),
        out_shape=jax.ShapeDtypeStruct((Np, 1), _f32),
    )(agg2p_t, y2, degp_t, b2)


@jax.jit
def kernel(x, edge_index, W1, b1, W2, b2):
    N, DI = x.shape
    DH = W1.shape[1]
    E = edge_index.shape[1]

    Np = ((N + 1 + NS * L - 1) // (NS * L)) * (NS * L)
    RING = 2 * NB
    nch = (E + CH - 1) // CH
    PW = (nch + NS - 1) // NS
    PW = (PW + RING - 1) // RING * RING
    TCH = NS * PW
    Ep = TCH * CH
    RCPW0 = max(RING, (PW * 30 // 100 + RING - 1) // RING * RING)
    RCPW1 = PW - RCPW0
    SCPW0 = max(1, PW * 4 // 10)
    SCPW1 = PW - SCPW0

    pad = jnp.full((Ep - E,), N, jnp.int32)
    src2 = jnp.concatenate([edge_index[0], pad]).reshape(TCH, CH)
    dst2 = jnp.concatenate([edge_index[1], pad]).reshape(TCH, CH)
    x_p = jnp.pad(x, ((0, Np - N), (0, 0)))

    degp = _scalar_agg(Np, SCPW1, SCPW0, with_table=False)(dst2)
    degp_t = degp.T
    y1 = _tc_y1(x_p, W1, degp_t)
    aggp = _row_agg(Np, DH, RCPW1, RCPW0)(y1, src2, dst2)
    y2 = _tc_y2(y1, aggp, degp_t, W2, b1.reshape(1, DH))
    agg2p = _scalar_agg(Np, SCPW1, SCPW0, with_table=True)(
        src2, dst2, y2.reshape(Np))
    out = _tc_out(agg2p.T, y2, degp_t, b2.reshape(1, 1))
    return out[:N]
